# Initial kernel scaffold; baseline (speedup 1.0000x reference)
import jax
import jax.numpy as jnp
from jax.experimental import pallas as pl


def kernel(x, router_w, conv_w, conv_b):
    raise NotImplementedError("write your pallas kernel here")


# speedup vs baseline: 1.1133x; 1.1133x over previous
import functools

import jax
import jax.numpy as jnp
from jax import lax
from jax.experimental import pallas as pl
from jax.experimental.pallas import tpu as pltpu

INPUT_DIM = 4096
PATCH_NUM = 16
KSZ = INPUT_DIM // PATCH_NUM
OUT_CHANNEL = 192
C2 = 2 * OUT_CHANNEL
EXPERT_NUM = 8
B = 2048
TB = 128
NB = B // TB


def _moe_body(x3_ref, wr_ref, wt_ref, b_ref, out_ref, sel0_ref, lb_ref,
              acc_ref):
    i = pl.program_id(0)
    rows = TB * PATCH_NUM

    x3 = x3_ref[...]
    xr = x3.reshape(TB, PATCH_NUM, KSZ)
    ps = xr[:, 0, :]
    for p_i in range(1, PATCH_NUM):
        ps = ps + xr[:, p_i, :]
    select = jnp.dot(ps.astype(jnp.bfloat16),
                     wr_ref[...].astype(jnp.bfloat16),
                     preferred_element_type=jnp.float32)

    gate = jnp.max(select, axis=1, keepdims=True)
    iota8 = lax.broadcasted_iota(jnp.int32, (TB, EXPERT_NUM), 1)
    idx = jnp.min(jnp.where(select == gate, iota8, EXPERT_NUM), axis=1,
                  keepdims=True)
    mask = (iota8 == idx).astype(jnp.float32)
    sel0_ref[...] = mask * (gate != 0.0).astype(jnp.float32)

    psel = jnp.sum(select, axis=0, keepdims=True)
    pmask = jnp.sum(mask, axis=0, keepdims=True)

    @pl.when(i == 0)
    def _init():
        acc_ref[...] = jnp.zeros_like(acc_ref)

    acc_ref[0:1, :] += psel
    acc_ref[1:2, :] += pmask

    iota_c = lax.broadcasted_iota(jnp.int32, (C2, 2), 0)
    iota_h = lax.broadcasted_iota(jnp.int32, (C2, 2), 1)
    sel_mat = ((iota_c < OUT_CHANNEL) == (iota_h == 0)).astype(jnp.float32)

    logits = jnp.zeros((TB, 2), dtype=jnp.float32)
    for e in range(EXPERT_NUM):
        m = jnp.dot(x3, wt_ref[e], preferred_element_type=jnp.float32)
        hh = m + b_ref[e, :][None, :]
        h3 = hh * hh * hh
        s2 = jnp.dot(h3, sel_mat, preferred_element_type=jnp.float32)
        sp = s2.reshape(TB, PATCH_NUM, 2).sum(axis=1)
        logits = logits + (gate * mask[:, e:e + 1]) * sp

    mx = jnp.max(logits, axis=1, keepdims=True)
    p = jnp.exp(logits - mx)
    out_ref[...] = p / jnp.sum(p, axis=1, keepdims=True)

    @pl.when(i == NB - 1)
    def _final():
        s = acc_ref[0:1, :] * acc_ref[1:2, :]
        lb = jnp.sum(s) * (float(EXPERT_NUM ** 2) /
                           (float(B) * float(B) * float(EXPERT_NUM)))
        lb_ref[...] = jnp.broadcast_to(lb, (1, 1))


@functools.partial(jax.jit, static_argnames=("interpret",))
def _moe_call(x3, wr_full, conv_wt, conv_b2, interpret=False):
    out, sel0, lb = pl.pallas_call(
        _moe_body,
        grid=(NB,),
        in_specs=[
            pl.BlockSpec((TB * PATCH_NUM, KSZ), lambda i: (i, 0)),
            pl.BlockSpec((KSZ, EXPERT_NUM), lambda i: (0, 0)),
            pl.BlockSpec((EXPERT_NUM, KSZ, C2), lambda i: (0, 0, 0)),
            pl.BlockSpec((EXPERT_NUM, C2), lambda i: (0, 0)),
        ],
        out_specs=[
            pl.BlockSpec((TB, 2), lambda i: (i, 0)),
            pl.BlockSpec((TB, EXPERT_NUM), lambda i: (i, 0)),
            pl.BlockSpec((1, 1), lambda i: (0, 0)),
        ],
        out_shape=[
            jax.ShapeDtypeStruct((B, 2), jnp.float32),
            jax.ShapeDtypeStruct((B, EXPERT_NUM), jnp.float32),
            jax.ShapeDtypeStruct((1, 1), jnp.float32),
        ],
        scratch_shapes=[pltpu.VMEM((2, EXPERT_NUM), jnp.float32)],
        interpret=interpret,
    )(x3, wr_full, conv_wt, conv_b2)
    return out, sel0, lb


def kernel(x, router_w, conv_w, conv_b, interpret=False):
    x3 = x.reshape(B * PATCH_NUM, KSZ)
    wr_full = router_w[:, 0, :].T
    conv_wt = conv_w[:, :, 0, :].transpose(0, 2, 1)
    out, sel0, lb = _moe_call(x3, wr_full, conv_wt, conv_b, interpret=interpret)
    return out, sel0, lb.reshape(())
